# baseline (speedup 1.0000x reference)
import functools

import jax
import jax.numpy as jnp
from jax import lax
from jax.experimental import pallas as pl
from jax.experimental.pallas import tpu as pltpu
from jax.experimental.pallas import tpu_sc as plsc

LAMBDA = 1.0

_NC = 2
_NS = 16
_NW = _NC * _NS
_L = 16

_ROWS = 16384
_D = 128
_RPW = _ROWS // _NW
_CR = 128
_CHUNKS = _D // _L


def _make_sc_partials():
    mesh = plsc.VectorSubcoreMesh(core_axis_name="c", subcore_axis_name="s")

    @functools.partial(
        pl.kernel,
        mesh=mesh,
        out_type=jax.ShapeDtypeStruct((_NW, _L), jnp.float32),
        scratch_types=[
            pltpu.VMEM((_CR, _D), jnp.float32),
            pltpu.VMEM((_CR, _D), jnp.float32),
            pltpu.VMEM((_RPW,), jnp.int32),
            pltpu.VMEM((1, _D), jnp.float32),
            pltpu.VMEM((1, _D), jnp.float32),
            pltpu.VMEM((_L,), jnp.float32),
            pltpu.SemaphoreType.DMA,
            pltpu.SemaphoreType.DMA,
        ],
    )
    def sc_partials(feat_hbm, lab_hbm, c0_hbm, c1_hbm, out_hbm,
                    buf0_v, buf1_v, lab_v, c0_v, c1_v, acc_v,
                    sem0, sem1):
        wid = lax.axis_index("s") * _NC + lax.axis_index("c")
        base = wid * _RPW
        nchunks = _RPW // _CR
        bufs = [buf0_v, buf1_v]
        sems = [sem0, sem1]
        handles = [
            pltpu.async_copy(
                feat_hbm.at[pl.ds(base + c * _CR, _CR)], bufs[c], sems[c])
            for c in range(2)
        ]
        pltpu.sync_copy(lab_hbm.at[pl.ds(base, _RPW)], lab_v)
        pltpu.sync_copy(c0_hbm, c0_v)
        pltpu.sync_copy(c1_hbm, c1_v)

        c0 = [c0_v[0, pl.ds(j * _L, _L)] for j in range(_CHUNKS)]
        dlt = [c1_v[0, pl.ds(j * _L, _L)] - c0[j] for j in range(_CHUNKS)]

        acc = jnp.zeros((_L,), jnp.float32)
        for c in range(nchunks):
            handles[c % 2].wait()
            buf = bufs[c % 2]

            def group_body(g, acc, c=c, buf=buf):
                lvf = lab_v[pl.ds(c * _CR + g * _L, _L)].astype(
                    jnp.float32)
                for k in range(_L):
                    lf = lvf[k]
                    for j in range(_CHUNKS):
                        t = (buf[g * _L + k, pl.ds(j * _L, _L)]
                             - c0[j] - lf * dlt[j])
                        acc = acc + t * t
                return acc

            acc = lax.fori_loop(0, _CR // _L, group_body, acc)
            if c + 2 < nchunks:
                handles[c % 2] = pltpu.async_copy(
                    feat_hbm.at[pl.ds(base + (c + 2) * _CR, _CR)],
                    buf, sems[c % 2])

        acc_v[...] = acc * (LAMBDA * 0.5 / _ROWS)
        pltpu.sync_copy(acc_v, out_hbm.at[wid])

    return sc_partials


_sc_partials = _make_sc_partials()


def kernel(features, labels, proto_0, proto_1):
    partials = _sc_partials(features, labels.astype(jnp.int32),
                            proto_0, proto_1)
    return jnp.sum(partials)
